# baseline (speedup 1.0000x reference)
import functools

import jax
import jax.numpy as jnp
from jax.experimental import pallas as pl
from jax.experimental.pallas import tpu as pltpu

N = 10000
D = 128


def _lin_kernel(x_ref, w_ref, zs_ref, buf_ref):
    acc = jnp.dot(x_ref[...].astype(jnp.bfloat16), w_ref[...],
                  preferred_element_type=jnp.float32)
    zs_ref[...] = acc.astype(jnp.bfloat16)


def _lin(x, w_cat):
    bm = 2000
    return pl.pallas_call(
        _lin_kernel,
        grid=(N // bm,),
        in_specs=[
            pl.BlockSpec((bm, D), lambda m: (m, 0)),
            pl.BlockSpec((D, 2 * D), lambda m: (0, 0)),
        ],
        out_specs=[
            pl.BlockSpec((bm, 2 * D), lambda m: (m, 0)),
            pl.BlockSpec((8, 3 * D), lambda m: (0, 0)),
        ],
        out_shape=[
            jax.ShapeDtypeStruct((N, 2 * D), jnp.bfloat16),
            jax.ShapeDtypeStruct((N, 3 * D), jnp.float32),
        ],
    )(x, w_cat)


def _agg_kernel(adj_ref, z_ref, s_ref, *rest, first, has_next):
    idx = 0
    w_ref = rest[idx] if has_next else None
    idx += has_next
    idx += 1
    h_ref = rest[idx]
    idx += 1
    zs_next_ref = rest[idx] if has_next else None
    idx += has_next
    adjb_ref = rest[idx] if first else None

    a = adj_ref[...]
    if first:
        a = a.astype(jnp.bfloat16)
        adjb_ref[...] = a
    acc = jnp.dot(a, z_ref[...], preferred_element_type=jnp.float32)
    h = jnp.maximum(acc + s_ref[...].astype(jnp.float32), 0.0)
    h_ref[...] = h
    if has_next:
        zs_next_ref[...] = jnp.dot(
            h.astype(jnp.bfloat16), w_ref[...],
            preferred_element_type=jnp.float32).astype(jnp.bfloat16)


def _agg(adj, zs, w_next, buf, layer, *, bm):
    first = layer == 0
    has_next = w_next is not None

    in_specs = [
        pl.BlockSpec((bm, N), lambda m: (m, 0)),
        pl.BlockSpec((N, D), lambda m: (0, 0)),
        pl.BlockSpec((bm, D), lambda m: (m, 1)),
    ]
    operands = [adj, zs, zs]
    if has_next:
        in_specs.append(pl.BlockSpec((D, 2 * D), lambda m: (0, 0)))
        operands.append(w_next)
    buf_in_idx = len(operands)
    in_specs.append(pl.BlockSpec((8, 3 * D), lambda m: (0, 0)))
    operands.append(buf)

    out_specs = [pl.BlockSpec((bm, D), lambda m, _l=layer: (m, _l))]
    out_shape = [jax.ShapeDtypeStruct((N, 3 * D), jnp.float32)]
    if has_next:
        out_specs.append(pl.BlockSpec((bm, 2 * D), lambda m: (m, 0)))
        out_shape.append(jax.ShapeDtypeStruct((N, 2 * D), jnp.bfloat16))
    if first:
        out_specs.append(pl.BlockSpec((bm, N), lambda m: (m, 0)))
        out_shape.append(jax.ShapeDtypeStruct((N, N), jnp.bfloat16))

    return pl.pallas_call(
        functools.partial(_agg_kernel, first=first, has_next=has_next),
        grid=(pl.cdiv(N, bm),),
        in_specs=in_specs,
        out_specs=out_specs,
        out_shape=out_shape,
        input_output_aliases={buf_in_idx: 0},
        compiler_params=pltpu.CompilerParams(
            dimension_semantics=("arbitrary",),
            vmem_limit_bytes=63 * 1024 * 1024),
    )(*operands)


def kernel(adj, x, Wn1, Ws1, Wn2, Ws2, Wn3, Ws3):
    w1 = jnp.concatenate([Wn1, Ws1], axis=1).astype(jnp.bfloat16)
    w2 = jnp.concatenate([Wn2, Ws2], axis=1).astype(jnp.bfloat16)
    w3 = jnp.concatenate([Wn3, Ws3], axis=1).astype(jnp.bfloat16)

    zs1, buf = _lin(x, w1)
    buf, zs2, adj_bf = _agg(adj, zs1, w2, buf, 0, bm=496)
    buf, zs3 = _agg(adj_bf, zs2, w3, buf, 1, bm=1240)
    (buf,) = _agg(adj_bf, zs3, None, buf, 2, bm=1240)
    return buf


# speedup vs baseline: 1.0301x; 1.0301x over previous
import functools

import jax
import jax.numpy as jnp
from jax.experimental import pallas as pl
from jax.experimental.pallas import tpu as pltpu

N = 10000
D = 128


def _lin_kernel(x_ref, w_ref, zs_ref, buf_ref):
    acc = jnp.dot(x_ref[...].astype(jnp.bfloat16), w_ref[...],
                  preferred_element_type=jnp.float32)
    zs_ref[...] = acc.astype(jnp.bfloat16)


def _lin(x, w_cat):
    bm = 2000
    return pl.pallas_call(
        _lin_kernel,
        grid=(N // bm,),
        in_specs=[
            pl.BlockSpec((bm, D), lambda m: (m, 0)),
            pl.BlockSpec((D, 2 * D), lambda m: (0, 0)),
        ],
        out_specs=[
            pl.BlockSpec((bm, 2 * D), lambda m: (m, 0)),
            pl.BlockSpec((8, 3 * D), lambda m: (0, 0)),
        ],
        out_shape=[
            jax.ShapeDtypeStruct((N, 2 * D), jnp.bfloat16),
            jax.ShapeDtypeStruct((N, 3 * D), jnp.float32),
        ],
    )(x, w_cat)


def _agg_kernel(adj_ref, z_ref, s_ref, *rest, first, has_next):
    idx = 0
    w_ref = rest[idx] if has_next else None
    idx += has_next
    idx += 1
    h_ref = rest[idx]
    idx += 1
    zs_next_ref = rest[idx] if has_next else None
    idx += has_next
    adjb_ref = rest[idx] if first else None

    a = adj_ref[...]
    if first:
        a = a.astype(jnp.bfloat16)
        adjb_ref[...] = a
    acc = jnp.dot(a, z_ref[...], preferred_element_type=jnp.float32)
    h = jnp.maximum(acc + s_ref[...].astype(jnp.float32), 0.0)
    h_ref[...] = h
    if has_next:
        zs_next_ref[...] = jnp.dot(
            h.astype(jnp.bfloat16), w_ref[...],
            preferred_element_type=jnp.float32).astype(jnp.bfloat16)


def _agg(adj, zs, w_next, buf, layer, *, bm):
    first = layer == 0
    has_next = w_next is not None

    in_specs = [
        pl.BlockSpec((bm, N), lambda m: (m, 0)),
        pl.BlockSpec((N, D), lambda m: (0, 0)),
        pl.BlockSpec((bm, D), lambda m: (m, 1)),
    ]
    operands = [adj, zs, zs]
    if has_next:
        in_specs.append(pl.BlockSpec((D, 2 * D), lambda m: (0, 0)))
        operands.append(w_next)
    buf_in_idx = len(operands)
    in_specs.append(pl.BlockSpec((8, 3 * D), lambda m: (0, 0)))
    operands.append(buf)

    out_specs = [pl.BlockSpec((bm, D), lambda m, _l=layer: (m, _l))]
    out_shape = [jax.ShapeDtypeStruct((N, 3 * D), jnp.float32)]
    if has_next:
        out_specs.append(pl.BlockSpec((bm, 2 * D), lambda m: (m, 0)))
        out_shape.append(jax.ShapeDtypeStruct((N, 2 * D), jnp.bfloat16))
    if first:
        out_specs.append(pl.BlockSpec((bm, N), lambda m: (m, 0)))
        out_shape.append(jax.ShapeDtypeStruct((N, N), jnp.bfloat16))

    return pl.pallas_call(
        functools.partial(_agg_kernel, first=first, has_next=has_next),
        grid=(pl.cdiv(N, bm),),
        in_specs=in_specs,
        out_specs=out_specs,
        out_shape=out_shape,
        input_output_aliases={buf_in_idx: 0},
        compiler_params=pltpu.CompilerParams(
            dimension_semantics=("arbitrary",),
            vmem_limit_bytes=63 * 1024 * 1024),
    )(*operands)


def kernel(adj, x, Wn1, Ws1, Wn2, Ws2, Wn3, Ws3):
    w1 = jnp.concatenate([Wn1, Ws1], axis=1).astype(jnp.bfloat16)
    w2 = jnp.concatenate([Wn2, Ws2], axis=1).astype(jnp.bfloat16)
    w3 = jnp.concatenate([Wn3, Ws3], axis=1).astype(jnp.bfloat16)

    zs1, buf = _lin(x, w1)
    buf, zs2, adj_bf = _agg(adj, zs1, w2, buf, 0, bm=496)
    buf, zs3 = _agg(adj_bf, zs2, w3, buf, 1, bm=1280)
    (buf,) = _agg(adj_bf, zs3, None, buf, 2, bm=1280)
    return buf


# speedup vs baseline: 1.0348x; 1.0045x over previous
import functools

import jax
import jax.numpy as jnp
from jax.experimental import pallas as pl
from jax.experimental.pallas import tpu as pltpu

N = 10000
D = 128


def _lin_kernel(x_ref, w_ref, zs_ref, buf_ref):
    acc = jnp.dot(x_ref[...].astype(jnp.bfloat16), w_ref[...],
                  preferred_element_type=jnp.float32)
    zs_ref[...] = acc.astype(jnp.bfloat16)


def _lin(x, w_cat):
    bm = 2000
    return pl.pallas_call(
        _lin_kernel,
        grid=(N // bm,),
        in_specs=[
            pl.BlockSpec((bm, D), lambda m: (m, 0)),
            pl.BlockSpec((D, 2 * D), lambda m: (0, 0)),
        ],
        out_specs=[
            pl.BlockSpec((bm, 2 * D), lambda m: (m, 0)),
            pl.BlockSpec((8, 3 * D), lambda m: (0, 0)),
        ],
        out_shape=[
            jax.ShapeDtypeStruct((N, 2 * D), jnp.bfloat16),
            jax.ShapeDtypeStruct((N, 3 * D), jnp.float32),
        ],
    )(x, w_cat)


def _agg_kernel(adj_ref, z_ref, s_ref, *rest, first, has_next):
    idx = 0
    w_ref = rest[idx] if has_next else None
    idx += has_next
    idx += 1
    h_ref = rest[idx]
    idx += 1
    zs_next_ref = rest[idx] if has_next else None
    idx += has_next
    adjb_ref = rest[idx] if first else None

    a = adj_ref[...]
    if first:
        a = a.astype(jnp.bfloat16)
        adjb_ref[...] = a
    acc = jnp.dot(a, z_ref[...], preferred_element_type=jnp.float32)
    h = jnp.maximum(acc + s_ref[...].astype(jnp.float32), 0.0)
    h_ref[...] = h
    if has_next:
        zs_next_ref[...] = jnp.dot(
            h.astype(jnp.bfloat16), w_ref[...],
            preferred_element_type=jnp.float32).astype(jnp.bfloat16)


def _agg(adj, zs, w_next, buf, layer, *, bm):
    first = layer == 0
    has_next = w_next is not None

    in_specs = [
        pl.BlockSpec((bm, N), lambda m: (m, 0)),
        pl.BlockSpec((N, D), lambda m: (0, 0)),
        pl.BlockSpec((bm, D), lambda m: (m, 1)),
    ]
    operands = [adj, zs, zs]
    if has_next:
        in_specs.append(pl.BlockSpec((D, 2 * D), lambda m: (0, 0)))
        operands.append(w_next)
    buf_in_idx = len(operands)
    in_specs.append(pl.BlockSpec((8, 3 * D), lambda m: (0, 0)))
    operands.append(buf)

    out_specs = [pl.BlockSpec((bm, D), lambda m, _l=layer: (m, _l))]
    out_shape = [jax.ShapeDtypeStruct((N, 3 * D), jnp.float32)]
    if has_next:
        out_specs.append(pl.BlockSpec((bm, 2 * D), lambda m: (m, 0)))
        out_shape.append(jax.ShapeDtypeStruct((N, 2 * D), jnp.bfloat16))
    if first:
        out_specs.append(pl.BlockSpec((bm, N), lambda m: (m, 0)))
        out_shape.append(jax.ShapeDtypeStruct((N, N), jnp.bfloat16))

    return pl.pallas_call(
        functools.partial(_agg_kernel, first=first, has_next=has_next),
        grid=(pl.cdiv(N, bm),),
        in_specs=in_specs,
        out_specs=out_specs,
        out_shape=out_shape,
        input_output_aliases={buf_in_idx: 0},
        compiler_params=pltpu.CompilerParams(
            dimension_semantics=("arbitrary",),
            vmem_limit_bytes=63 * 1024 * 1024),
    )(*operands)


def kernel(adj, x, Wn1, Ws1, Wn2, Ws2, Wn3, Ws3):
    w1 = jnp.concatenate([Wn1, Ws1], axis=1).astype(jnp.bfloat16)
    w2 = jnp.concatenate([Wn2, Ws2], axis=1).astype(jnp.bfloat16)
    w3 = jnp.concatenate([Wn3, Ws3], axis=1).astype(jnp.bfloat16)

    zs1, buf = _lin(x, w1)
    buf, zs2, adj_bf = _agg(adj, zs1, w2, buf, 0, bm=448)
    buf, zs3 = _agg(adj_bf, zs2, w3, buf, 1, bm=1280)
    (buf,) = _agg(adj_bf, zs3, None, buf, 2, bm=1280)
    return buf


# speedup vs baseline: 1.0358x; 1.0010x over previous
import functools

import jax
import jax.numpy as jnp
from jax.experimental import pallas as pl
from jax.experimental.pallas import tpu as pltpu

N = 10000
D = 128


def _lin_kernel(x_ref, w_ref, zs_ref, buf_ref):
    acc = jnp.dot(x_ref[...].astype(jnp.bfloat16), w_ref[...],
                  preferred_element_type=jnp.float32)
    zs_ref[...] = acc.astype(jnp.bfloat16)


def _lin(x, w_cat):
    bm = 2000
    return pl.pallas_call(
        _lin_kernel,
        grid=(N // bm,),
        in_specs=[
            pl.BlockSpec((bm, D), lambda m: (m, 0)),
            pl.BlockSpec((D, 2 * D), lambda m: (0, 0)),
        ],
        out_specs=[
            pl.BlockSpec((bm, 2 * D), lambda m: (m, 0)),
            pl.BlockSpec((8, 3 * D), lambda m: (0, 0)),
        ],
        out_shape=[
            jax.ShapeDtypeStruct((N, 2 * D), jnp.bfloat16),
            jax.ShapeDtypeStruct((N, 3 * D), jnp.float32),
        ],
    )(x, w_cat)


def _agg_kernel(adj_ref, z_ref, s_ref, *rest, first, has_next):
    idx = 0
    w_ref = rest[idx] if has_next else None
    idx += has_next
    idx += 1
    h_ref = rest[idx]
    idx += 1
    zs_next_ref = rest[idx] if has_next else None
    idx += has_next
    adjb_ref = rest[idx] if first else None

    a = adj_ref[...]
    if first:
        a = a.astype(jnp.bfloat16)
        adjb_ref[...] = a
    acc = jnp.dot(a, z_ref[...], preferred_element_type=jnp.float32)
    h = jnp.maximum(acc + s_ref[...].astype(jnp.float32), 0.0)
    h_ref[...] = h
    if has_next:
        zs_next_ref[...] = jnp.dot(
            h.astype(jnp.bfloat16), w_ref[...],
            preferred_element_type=jnp.float32).astype(jnp.bfloat16)


def _agg(adj, zs, w_next, buf, layer, *, bm):
    first = layer == 0
    has_next = w_next is not None

    in_specs = [
        pl.BlockSpec((bm, N), lambda m: (m, 0)),
        pl.BlockSpec((N, D), lambda m: (0, 0)),
        pl.BlockSpec((bm, D), lambda m: (m, 1)),
    ]
    operands = [adj, zs, zs]
    if has_next:
        in_specs.append(pl.BlockSpec((D, 2 * D), lambda m: (0, 0)))
        operands.append(w_next)
    buf_in_idx = len(operands)
    in_specs.append(pl.BlockSpec((8, 3 * D), lambda m: (0, 0)))
    operands.append(buf)

    out_specs = [pl.BlockSpec((bm, D), lambda m, _l=layer: (m, _l))]
    out_shape = [jax.ShapeDtypeStruct((N, 3 * D), jnp.float32)]
    if has_next:
        out_specs.append(pl.BlockSpec((bm, 2 * D), lambda m: (m, 0)))
        out_shape.append(jax.ShapeDtypeStruct((N, 2 * D), jnp.bfloat16))
    if first:
        out_specs.append(pl.BlockSpec((bm, N), lambda m: (m, 0)))
        out_shape.append(jax.ShapeDtypeStruct((N, N), jnp.bfloat16))

    return pl.pallas_call(
        functools.partial(_agg_kernel, first=first, has_next=has_next),
        grid=(pl.cdiv(N, bm),),
        in_specs=in_specs,
        out_specs=out_specs,
        out_shape=out_shape,
        input_output_aliases={buf_in_idx: 0},
        compiler_params=pltpu.CompilerParams(
            dimension_semantics=("arbitrary",),
            vmem_limit_bytes=63 * 1024 * 1024),
    )(*operands)


def kernel(adj, x, Wn1, Ws1, Wn2, Ws2, Wn3, Ws3):
    w1 = jnp.concatenate([Wn1, Ws1], axis=1).astype(jnp.bfloat16)
    w2 = jnp.concatenate([Wn2, Ws2], axis=1).astype(jnp.bfloat16)
    w3 = jnp.concatenate([Wn3, Ws3], axis=1).astype(jnp.bfloat16)

    zs1, buf = _lin(x, w1)
    buf, zs2, adj_bf = _agg(adj, zs1, w2, buf, 0, bm=384)
    buf, zs3 = _agg(adj_bf, zs2, w3, buf, 1, bm=1280)
    (buf,) = _agg(adj_bf, zs3, None, buf, 2, bm=1344)
    return buf
